# Initial kernel scaffold; baseline (speedup 1.0000x reference)
import jax
import jax.numpy as jnp
from jax.experimental import pallas as pl


def kernel(h, e, edge_index, params):
    raise NotImplementedError("write your pallas kernel here")


# speedup vs baseline: 1.2200x; 1.2200x over previous
import functools

import jax
import jax.numpy as jnp
from jax import lax
from jax.experimental import pallas as pl
from jax.experimental.pallas import tpu as pltpu
from jax.experimental.pallas import tpu_sc as plsc

N = 10000
E = 320000
D = 128
H = 64

BN = 1000
NB_N = N // BN
BE = 2000
NB_E = E // BE

NTILE = 16
EPT = E // NTILE
BSC = 80
NBLK = EPT // BSC


_EPS_DEN = 1e-6
_EPS_BN = 1e-5


def _node_mm_body(h_ref, uw, ub, vw, vb, aw, ab, bw, bb,
                  uh_ref, tdst_ref, tsrc_ref):
    hb = h_ref[...]
    uh_ref[...] = hb @ uw[...] + ub[...]
    tdst_ref[...] = hb @ aw[...] + ab[...]
    eh = hb @ bw[...] + bb[...]
    vh = hb @ vw[...] + vb[...]
    tsrc_ref[0] = jnp.concatenate([eh[:, :H], vh[:, :H]], axis=1)
    tsrc_ref[1] = jnp.concatenate([eh[:, H:], vh[:, H:]], axis=1)


def _node_mm(h, p):
    w = lambda: pl.BlockSpec((D, D), lambda i: (0, 0))
    b = lambda: pl.BlockSpec((1, D), lambda i: (0, 0))
    return pl.pallas_call(
        _node_mm_body,
        grid=(NB_N,),
        in_specs=[pl.BlockSpec((BN, D), lambda i: (i, 0)),
                  w(), b(), w(), b(), w(), b(), w(), b()],
        out_specs=[pl.BlockSpec((BN, D), lambda i: (i, 0)),
                   pl.BlockSpec((BN, D), lambda i: (i, 0)),
                   pl.BlockSpec((2, BN, D), lambda i: (0, i, 0))],
        out_shape=[jax.ShapeDtypeStruct((N, D), jnp.float32),
                   jax.ShapeDtypeStruct((N, D), jnp.float32),
                   jax.ShapeDtypeStruct((2, N, D), jnp.float32)],
    )(h, p["U_w"], p["U_b"].reshape(1, D), p["V_w"], p["V_b"].reshape(1, D),
      p["A_w"], p["A_b"].reshape(1, D), p["B_w"], p["B_b"].reshape(1, D))


def _edge_mm_body(e_ref, cw, cb, ce_ref):
    ce = e_ref[...] @ cw[...] + cb[...]
    ce_ref[0] = ce[:, :H]
    ce_ref[1] = ce[:, H:]


def _edge_mm(e, p):
    return pl.pallas_call(
        _edge_mm_body,
        grid=(NB_E,),
        in_specs=[pl.BlockSpec((BE, D), lambda i: (i, 0)),
                  pl.BlockSpec((D, D), lambda i: (0, 0)),
                  pl.BlockSpec((1, D), lambda i: (0, 0))],
        out_specs=[pl.BlockSpec((2, BE, H), lambda i: (0, i, 0))],
        out_shape=[jax.ShapeDtypeStruct((2, E, H), jnp.float32)],
    )(e, p["C_w"], p["C_b"].reshape(1, D))[0]


_sc_mesh = plsc.VectorSubcoreMesh(core_axis_name="c", subcore_axis_name="s")


@functools.partial(
    pl.kernel,
    mesh=_sc_mesh,
    out_type=[
        jax.ShapeDtypeStruct((2 * E, H), jnp.float32),
        jax.ShapeDtypeStruct((2, N, D), jnp.float32),
    ],
    scratch_types=[
        pltpu.VMEM((BSC,), jnp.int32),
        pltpu.VMEM((BSC,), jnp.int32),
        pltpu.VMEM((BSC,), jnp.int32),
        pltpu.VMEM((BSC, D), jnp.float32),
        pltpu.VMEM((BSC, D), jnp.float32),
        pltpu.VMEM((BSC, H), jnp.float32),
        pltpu.VMEM((BSC, D), jnp.float32),
        pltpu.VMEM_SHARED((N, D), jnp.float32),
        pltpu.SemaphoreType.DMA,
        pltpu.SemaphoreType.DMA,
    ],
)
def _sc_edge(tsrc_hbm, tdst_hbm, ce_hbm, src_hbm, dst_hbm,
             ehat_hbm, numden_hbm,
             src_v, dst_v, soff_v, tsrc_rows, tdst_rows, ce_rows,
             srow, accum, sem1, sem2):
    half = lax.axis_index("c")
    tile = lax.axis_index("s")

    def _zrow(i, _):
        for j in range(D // 16):
            srow[i, pl.ds(j * 16, 16)] = jnp.zeros((16,), jnp.float32)
        return 0
    lax.fori_loop(0, BSC, _zrow, 0)

    row0 = tile * 624
    nfull = jnp.where(tile == 15, 8, 7)

    def _zacc(k, _):
        pltpu.sync_copy(srow, accum.at[pl.ds(row0 + k * 80, 80)])
        return 0
    lax.fori_loop(0, nfull, _zacc, 0)

    @pl.when(tile < 15)
    def _():
        pltpu.sync_copy(srow.at[pl.ds(0, 64)],
                        accum.at[pl.ds(row0 + 560, 64)])

    plsc.subcore_barrier()

    off = half * N
    base0 = tile * EPT

    def _blk(b, _):
        base = base0 + b * BSC
        pltpu.sync_copy(src_hbm.at[pl.ds(base, BSC)], src_v)
        pltpu.sync_copy(dst_hbm.at[pl.ds(base, BSC)], dst_v)
        for j in range(BSC // 16):
            sl = pl.ds(j * 16, 16)
            soff_v[sl] = src_v[sl] + off
        g1 = pltpu.async_copy(tsrc_hbm.at[soff_v], tsrc_rows, sem1)
        g2 = pltpu.async_copy(tdst_hbm.at[dst_v], tdst_rows, sem2)
        pltpu.sync_copy(ce_hbm.at[pl.ds(half * E + base, BSC)], ce_rows)
        g1.wait()
        g2.wait()

        def _edge(i, _):
            for j in range(H // 16):
                sl = pl.ds(j * 16, 16)
                sv = pl.ds(H + j * 16, 16)
                x = (tdst_rows[i, pl.ds(half * H + j * 16, 16)]
                     + tsrc_rows[i, sl] + ce_rows[i, sl])
                sg = 1.0 / (1.0 + jnp.exp(-x))
                ce_rows[i, sl] = x
                srow[i, sl] = sg * tsrc_rows[i, sv]
                srow[i, sv] = sg
            return 0
        lax.fori_loop(0, BSC, _edge, 0)

        pltpu.sync_copy(ce_rows, ehat_hbm.at[pl.ds(half * E + base, BSC)])
        pltpu.sync_copy(srow, accum.at[dst_v], add=True)
        return 0

    lax.fori_loop(0, NBLK, _blk, 0)
    plsc.subcore_barrier()

    def _cpo(k, _):
        r0 = row0 + k * 80
        pltpu.sync_copy(accum.at[pl.ds(r0, 80)],
                        numden_hbm.at[half, pl.ds(r0, 80)])
        return 0
    lax.fori_loop(0, nfull, _cpo, 0)

    @pl.when(tile < 15)
    def _():
        pltpu.sync_copy(accum.at[pl.ds(row0 + 560, 64)],
                        numden_hbm.at[half, pl.ds(row0 + 560, 64)])


def _node_out_body(uh_ref, nd_ref, h_ref, out_ref, hh_scr, st_scr):
    i = pl.program_id(0)

    @pl.when(i < NB_N)
    def _():
        num = jnp.concatenate([nd_ref[0, :, :H], nd_ref[1, :, :H]], axis=1)
        den = jnp.concatenate([nd_ref[0, :, H:], nd_ref[1, :, H:]], axis=1)
        hh = uh_ref[...] + num / (den + _EPS_DEN)
        hh_scr[pl.ds(i * BN, BN), :] = hh

        @pl.when(i == 0)
        def _():
            st_scr[...] = jnp.zeros_like(st_scr)
        st_scr[0:1, :] += jnp.sum(hh, axis=0, keepdims=True)
        st_scr[1:2, :] += jnp.sum(hh * hh, axis=0, keepdims=True)

    @pl.when(i >= NB_N)
    def _():
        k = i - NB_N
        mu = st_scr[0:1, :] / N
        var = st_scr[1:2, :] / N - mu * mu
        inv = lax.rsqrt(var + _EPS_BN)
        hh = hh_scr[pl.ds(k * BN, BN), :]
        out_ref[...] = h_ref[...] + jnp.maximum((hh - mu) * inv, 0.0)


def _node_out(uh, numden, h):
    return pl.pallas_call(
        _node_out_body,
        grid=(2 * NB_N,),
        in_specs=[pl.BlockSpec((BN, D), lambda i: (i % NB_N, 0)),
                  pl.BlockSpec((2, BN, D), lambda i: (0, i % NB_N, 0)),
                  pl.BlockSpec((BN, D), lambda i: (i % NB_N, 0))],
        out_specs=[pl.BlockSpec((BN, D), lambda i: (i % NB_N, 0))],
        out_shape=[jax.ShapeDtypeStruct((N, D), jnp.float32)],
        scratch_shapes=[pltpu.VMEM((N, D), jnp.float32),
                        pltpu.VMEM((2, D), jnp.float32)],
    )(uh, numden, h)[0]


def _edge_out_body(ehat_ref, e_ref, out_ref, st_scr):
    i = pl.program_id(0)

    @pl.when(i < NB_E)
    def _():
        b = ehat_ref[...]

        @pl.when(i == 0)
        def _():
            st_scr[...] = jnp.zeros_like(st_scr)
        st_scr[0] += jnp.sum(b, axis=1)
        st_scr[1] += jnp.sum(b * b, axis=1)

    @pl.when(i >= NB_E)
    def _():
        mu = st_scr[0] / E
        var = st_scr[1] / E - mu * mu
        inv = lax.rsqrt(var + _EPS_BN)
        b = ehat_ref[...]
        eb = e_ref[...]
        r0 = jnp.maximum((b[0] - mu[0:1, :]) * inv[0:1, :], 0.0)
        r1 = jnp.maximum((b[1] - mu[1:2, :]) * inv[1:2, :], 0.0)
        out_ref[...] = eb + jnp.concatenate([r0, r1], axis=1)


def _edge_out(ehat, e):
    return pl.pallas_call(
        _edge_out_body,
        grid=(2 * NB_E,),
        in_specs=[pl.BlockSpec((2, BE, H), lambda i: (0, i % NB_E, 0)),
                  pl.BlockSpec((BE, D), lambda i: (i % NB_E, 0))],
        out_specs=[pl.BlockSpec((BE, D), lambda i: (i % NB_E, 0))],
        out_shape=[jax.ShapeDtypeStruct((E, D), jnp.float32)],
        scratch_shapes=[pltpu.VMEM((2, 2, H), jnp.float32)],
    )(ehat, e)[0]


def kernel(h, e, edge_index, params):
    src = edge_index[0]
    dst = edge_index[1]
    for p in params:
        uh, tdst, tsrc = _node_mm(h, p)
        ce = _edge_mm(e, p)
        ehat2, numden = _sc_edge(
            tsrc.reshape(2 * N, D), tdst,
            ce.reshape(2 * E, H), src, dst)
        h = _node_out(uh, numden, h)
        e = _edge_out(ehat2.reshape(2, E, H), e)
    return (h, e)


# speedup vs baseline: 3.0216x; 2.4768x over previous
import functools

import jax
import jax.numpy as jnp
from jax import lax
from jax.experimental import pallas as pl
from jax.experimental.pallas import tpu as pltpu
from jax.experimental.pallas import tpu_sc as plsc

N = 10000
E = 320000
D = 128
H = 64

BN = 1000
NB_N = N // BN
BE = 2000
NB_E = E // BE

NTILE = 16
EPT = E // NTILE
BSC = 48
SBLK = 8
SEDGE = SBLK * BSC
NSUP = 52
NBLKS = NSUP * SBLK
TAIL = EPT - NBLKS * BSC


_EPS_DEN = 1e-6
_EPS_BN = 1e-5


def _node_mm_body(h_ref, uw, ub, vw, vb, aw, ab, bw, bb,
                  uh_ref, tdst_ref, tsrc_ref):
    hb = h_ref[...]
    uh_ref[...] = hb @ uw[...] + ub[...]
    tdst_ref[...] = hb @ aw[...] + ab[...]
    eh = hb @ bw[...] + bb[...]
    vh = hb @ vw[...] + vb[...]
    tsrc_ref[0] = jnp.concatenate([eh[:, :H], vh[:, :H]], axis=1)
    tsrc_ref[1] = jnp.concatenate([eh[:, H:], vh[:, H:]], axis=1)


def _node_mm(h, p):
    w = lambda: pl.BlockSpec((D, D), lambda i: (0, 0))
    b = lambda: pl.BlockSpec((1, D), lambda i: (0, 0))
    return pl.pallas_call(
        _node_mm_body,
        grid=(NB_N,),
        in_specs=[pl.BlockSpec((BN, D), lambda i: (i, 0)),
                  w(), b(), w(), b(), w(), b(), w(), b()],
        out_specs=[pl.BlockSpec((BN, D), lambda i: (i, 0)),
                   pl.BlockSpec((BN, D), lambda i: (i, 0)),
                   pl.BlockSpec((2, BN, D), lambda i: (0, i, 0))],
        out_shape=[jax.ShapeDtypeStruct((N, D), jnp.float32),
                   jax.ShapeDtypeStruct((N, D), jnp.float32),
                   jax.ShapeDtypeStruct((2, N, D), jnp.float32)],
    )(h, p["U_w"], p["U_b"].reshape(1, D), p["V_w"], p["V_b"].reshape(1, D),
      p["A_w"], p["A_b"].reshape(1, D), p["B_w"], p["B_b"].reshape(1, D))


def _edge_mm_body(e_ref, cw, cb, ce_ref):
    ce = e_ref[...] @ cw[...] + cb[...]
    ce_ref[0] = ce[:, :H]
    ce_ref[1] = ce[:, H:]


def _edge_mm(e, p):
    return pl.pallas_call(
        _edge_mm_body,
        grid=(NB_E,),
        in_specs=[pl.BlockSpec((BE, D), lambda i: (i, 0)),
                  pl.BlockSpec((D, D), lambda i: (0, 0)),
                  pl.BlockSpec((1, D), lambda i: (0, 0))],
        out_specs=[pl.BlockSpec((2, BE, H), lambda i: (0, i, 0))],
        out_shape=[jax.ShapeDtypeStruct((2, E, H), jnp.float32)],
    )(e, p["C_w"], p["C_b"].reshape(1, D))[0]


_sc_mesh = plsc.VectorSubcoreMesh(core_axis_name="c", subcore_axis_name="s")


@functools.partial(
    pl.kernel,
    mesh=_sc_mesh,
    out_type=[
        jax.ShapeDtypeStruct((2 * E, H), jnp.float32),
        jax.ShapeDtypeStruct((2, N, D), jnp.float32),
    ],
    scratch_types=[
        pltpu.VMEM((SEDGE,), jnp.int32),
        pltpu.VMEM((SEDGE,), jnp.int32),
        pltpu.VMEM((SEDGE,), jnp.int32),
        pltpu.VMEM((SEDGE,), jnp.int32),
        pltpu.VMEM((BSC,), jnp.int32),
        pltpu.VMEM((BSC,), jnp.int32),
        pltpu.VMEM((BSC,), jnp.int32),
        pltpu.VMEM((BSC,), jnp.int32),
        pltpu.VMEM((BSC, D), jnp.float32),
        pltpu.VMEM((BSC, D), jnp.float32),
        pltpu.VMEM((BSC, D), jnp.float32),
        pltpu.VMEM((BSC, D), jnp.float32),
        pltpu.VMEM((BSC, H), jnp.float32),
        pltpu.VMEM((BSC, H), jnp.float32),
        pltpu.VMEM((BSC, D), jnp.float32),
        pltpu.VMEM((TAIL,), jnp.int32),
        pltpu.VMEM((TAIL,), jnp.int32),
        pltpu.VMEM_SHARED((N, D), jnp.float32),
        pltpu.SemaphoreType.DMA,
        pltpu.SemaphoreType.DMA,
        pltpu.SemaphoreType.DMA,
        pltpu.SemaphoreType.DMA,
        pltpu.SemaphoreType.DMA,
        pltpu.SemaphoreType.DMA,
        pltpu.SemaphoreType.DMA,
        pltpu.SemaphoreType.DMA,
        pltpu.SemaphoreType.DMA,
        pltpu.SemaphoreType.DMA,
        pltpu.SemaphoreType.DMA,
        pltpu.SemaphoreType.DMA,
    ],
)
def _sc_edge(tsrc_hbm, tdst_hbm, ce_hbm, soff_hbm, dst_hbm,
             ehat_hbm, numden_hbm,
             idxs0, idxs1, idxd0, idxd1, sofS0, sofS1, dstS0, dstS1,
             tsr0, tsr1, tdr0, tdr1, cer0, cer1, srow,
             sofT, dstT, accum,
             si0, si1, di0, di1, ts0, ts1, td0, td1, ce0, ce1, eh0, eh1):
    half = lax.axis_index("c")
    tile = lax.axis_index("s")
    idxs = (idxs0, idxs1)
    idxd = (idxd0, idxd1)
    sofS = (sofS0, sofS1)
    dstS = (dstS0, dstS1)
    tsr = (tsr0, tsr1)
    tdr = (tdr0, tdr1)
    cer = (cer0, cer1)
    si = (si0, si1)
    di = (di0, di1)
    ts = (ts0, ts1)
    td = (td0, td1)
    cs = (ce0, ce1)
    eh = (eh0, eh1)

    def _zrow(i, _):
        for j in range(D // 16):
            srow[i, pl.ds(j * 16, 16)] = jnp.zeros((16,), jnp.float32)
        return 0
    lax.fori_loop(0, BSC, _zrow, 0)

    row0 = tile * 624

    def _zacc(k, _):
        pltpu.sync_copy(srow, accum.at[pl.ds(row0 + k * BSC, BSC)])
        return 0
    lax.fori_loop(0, 13, _zacc, 0)

    @pl.when(tile == 15)
    def _():
        pltpu.sync_copy(srow.at[pl.ds(0, 16)], accum.at[pl.ds(row0 + 624, 16)])
    plsc.subcore_barrier()

    base0 = tile * EPT
    soff0 = half * E

    def _idx_slices(dst_par, src_buf, off):
        for c in range(BSC // 16):
            sl = pl.ds(c * 16, 16)
            sofS[dst_par][sl] = idxs[src_buf][pl.ds(off + c * 16, 16)]
            dstS[dst_par][sl] = idxd[src_buf][pl.ds(off + c * 16, 16)]

    def _issue(par, b):
        ebase = base0 + b * BSC
        pltpu.async_copy(tsrc_hbm.at[sofS[par]], tsr[par], ts[par])
        pltpu.async_copy(tdst_hbm.at[dstS[par]], tdr[par], td[par])
        pltpu.async_copy(ce_hbm.at[pl.ds(half * E + ebase, BSC)],
                         cer[par], cs[par])

    def _wait(par, b):
        ebase = base0 + b * BSC
        pltpu.make_async_copy(tsrc_hbm.at[sofS[par]], tsr[par],
                              ts[par]).wait()
        pltpu.make_async_copy(tdst_hbm.at[dstS[par]], tdr[par],
                              td[par]).wait()
        pltpu.make_async_copy(ce_hbm.at[pl.ds(half * E + ebase, BSC)],
                              cer[par], cs[par]).wait()

    def _drain_ehat(par):
        pltpu.make_async_copy(cer[par],
                              ehat_hbm.at[pl.ds(half * E + base0, BSC)],
                              eh[par]).wait()

    def _compute(par, nedge):
        @plsc.parallel_loop(0, nedge, unroll=4)
        def _edge(i):
            for j in range(H // 16):
                sl = pl.ds(j * 16, 16)
                sv = pl.ds(H + j * 16, 16)
                x = (tdr[par][i, pl.ds(half * H + j * 16, 16)]
                     + tsr[par][i, sl] + cer[par][i, sl])
                sg = 1.0 / (1.0 + jnp.exp(-x))
                cer[par][i, sl] = x
                srow[i, sl] = sg * tsr[par][i, sv]
                srow[i, sv] = sg

    pltpu.sync_copy(soff_hbm.at[pl.ds(soff0 + base0, SEDGE)], idxs[0])
    pltpu.sync_copy(dst_hbm.at[pl.ds(base0, SEDGE)], idxd[0])
    _idx_slices(0, 0, 0)
    _issue(0, 0)

    def _sup_pair(sp, _):
        for p in range(2):
            sb = sp * 2 + p
            sbase = base0 + sb * SEDGE

            def _prefetch():
                nbase = sbase + SEDGE
                pltpu.async_copy(soff_hbm.at[pl.ds(soff0 + nbase, SEDGE)],
                                 idxs[1 - p], si[1 - p])
                pltpu.async_copy(dst_hbm.at[pl.ds(nbase, SEDGE)],
                                 idxd[1 - p], di[1 - p])
            if p == 0:
                _prefetch()
            else:
                pl.when(sp < NSUP // 2 - 1)(_prefetch)

            def _q(q, _):
                for u in range(2):
                    b = sb * SBLK + q * 2 + u
                    _wait(u, b)

                    def _stage_cur():
                        _idx_slices(1 - u, p, (q * 2 + u + 1) * BSC)

                    def _stage_pref():
                        nbase2 = sbase + SEDGE
                        pltpu.make_async_copy(
                            soff_hbm.at[pl.ds(soff0 + nbase2, SEDGE)],
                            idxs[1 - p], si[1 - p]).wait()
                        pltpu.make_async_copy(
                            dst_hbm.at[pl.ds(nbase2, SEDGE)],
                            idxd[1 - p], di[1 - p]).wait()
                        _idx_slices(1 - u, 1 - p, 0)

                    if u == 0:
                        _stage_cur()

                        @pl.when(b >= 1)
                        def _():
                            _drain_ehat(1)
                        _issue(1, b + 1)
                    else:
                        @pl.when(q < SBLK // 2 - 1)
                        def _():
                            _stage_cur()
                            _drain_ehat(0)
                            _issue(0, b + 1)

                        @pl.when(jnp.logical_and(q == SBLK // 2 - 1,
                                                 b + 1 < NBLKS))
                        def _():
                            _stage_pref()
                            _drain_ehat(0)
                            _issue(0, b + 1)

                    _compute(u, BSC)
                    ebase = base0 + b * BSC
                    pltpu.async_copy(cer[u],
                                     ehat_hbm.at[pl.ds(half * E + ebase, BSC)],
                                     eh[u])
                    pltpu.sync_copy(srow, accum.at[dstS[u]], add=True)
                return 0
            lax.fori_loop(0, SBLK // 2, _q, 0)
        return 0
    lax.fori_loop(0, NSUP // 2, _sup_pair, 0)

    _drain_ehat(0)
    _drain_ehat(1)

    tbase = base0 + NBLKS * BSC
    pltpu.sync_copy(soff_hbm.at[pl.ds(soff0 + tbase, TAIL)], sofT)
    pltpu.sync_copy(dst_hbm.at[pl.ds(tbase, TAIL)], dstT)
    t1 = pltpu.async_copy(tsrc_hbm.at[sofT], tsr0.at[pl.ds(0, TAIL)], ts0)
    t2 = pltpu.async_copy(tdst_hbm.at[dstT], tdr0.at[pl.ds(0, TAIL)], td0)
    pltpu.sync_copy(ce_hbm.at[pl.ds(half * E + tbase, TAIL)],
                    cer0.at[pl.ds(0, TAIL)])
    t1.wait()
    t2.wait()
    _compute(0, TAIL)
    pltpu.sync_copy(cer0.at[pl.ds(0, TAIL)],
                    ehat_hbm.at[pl.ds(half * E + tbase, TAIL)])
    pltpu.sync_copy(srow.at[pl.ds(0, TAIL)], accum.at[dstT], add=True)

    plsc.subcore_barrier()

    def _cpo(k, _):
        r0 = row0 + k * BSC
        pltpu.sync_copy(accum.at[pl.ds(r0, BSC)],
                        numden_hbm.at[half, pl.ds(r0, BSC)])
        return 0
    lax.fori_loop(0, 13, _cpo, 0)

    @pl.when(tile == 15)
    def _():
        pltpu.sync_copy(accum.at[pl.ds(row0 + 624, 16)],
                        numden_hbm.at[half, pl.ds(row0 + 624, 16)])


def _node_out_body(uh_ref, nd_ref, h_ref, out_ref, hh_scr, st_scr):
    i = pl.program_id(0)

    @pl.when(i < NB_N)
    def _():
        num = jnp.concatenate([nd_ref[0, :, :H], nd_ref[1, :, :H]], axis=1)
        den = jnp.concatenate([nd_ref[0, :, H:], nd_ref[1, :, H:]], axis=1)
        hh = uh_ref[...] + num / (den + _EPS_DEN)
        hh_scr[pl.ds(i * BN, BN), :] = hh

        @pl.when(i == 0)
        def _():
            st_scr[...] = jnp.zeros_like(st_scr)
        st_scr[0:1, :] += jnp.sum(hh, axis=0, keepdims=True)
        st_scr[1:2, :] += jnp.sum(hh * hh, axis=0, keepdims=True)

    @pl.when(i >= NB_N)
    def _():
        k = i - NB_N
        mu = st_scr[0:1, :] / N
        var = st_scr[1:2, :] / N - mu * mu
        inv = lax.rsqrt(var + _EPS_BN)
        hh = hh_scr[pl.ds(k * BN, BN), :]
        out_ref[...] = h_ref[...] + jnp.maximum((hh - mu) * inv, 0.0)


def _node_out(uh, numden, h):
    return pl.pallas_call(
        _node_out_body,
        grid=(2 * NB_N,),
        in_specs=[pl.BlockSpec((BN, D), lambda i: (i % NB_N, 0)),
                  pl.BlockSpec((2, BN, D), lambda i: (0, i % NB_N, 0)),
                  pl.BlockSpec((BN, D), lambda i: (i % NB_N, 0))],
        out_specs=[pl.BlockSpec((BN, D), lambda i: (i % NB_N, 0))],
        out_shape=[jax.ShapeDtypeStruct((N, D), jnp.float32)],
        scratch_shapes=[pltpu.VMEM((N, D), jnp.float32),
                        pltpu.VMEM((2, D), jnp.float32)],
    )(uh, numden, h)[0]


def _edge_stats(ehat_ref, st_scr, i):
    @pl.when(i < NB_E)
    def _():
        b = ehat_ref[...]

        @pl.when(i == 0)
        def _():
            st_scr[...] = jnp.zeros_like(st_scr)
        st_scr[0] += jnp.sum(b, axis=1)
        st_scr[1] += jnp.sum(b * b, axis=1)


def _edge_norm(ehat_ref, e_ref, st_scr):
    mu = st_scr[0] / E
    var = st_scr[1] / E - mu * mu
    inv = lax.rsqrt(var + _EPS_BN)
    b = ehat_ref[...]
    r0 = jnp.maximum((b[0] - mu[0:1, :]) * inv[0:1, :], 0.0)
    r1 = jnp.maximum((b[1] - mu[1:2, :]) * inv[1:2, :], 0.0)
    return e_ref[...] + jnp.concatenate([r0, r1], axis=1)


def _edge_out_body(ehat_ref, e_ref, out_ref, st_scr):
    i = pl.program_id(0)
    _edge_stats(ehat_ref, st_scr, i)

    @pl.when(i >= NB_E)
    def _():
        out_ref[...] = _edge_norm(ehat_ref, e_ref, st_scr)


def _edge_out(ehat, e):
    return pl.pallas_call(
        _edge_out_body,
        grid=(2 * NB_E,),
        in_specs=[pl.BlockSpec((2, BE, H), lambda i: (0, i % NB_E, 0)),
                  pl.BlockSpec((BE, D), lambda i: (i % NB_E, 0))],
        out_specs=[pl.BlockSpec((BE, D), lambda i: (i % NB_E, 0))],
        out_shape=[jax.ShapeDtypeStruct((E, D), jnp.float32)],
        scratch_shapes=[pltpu.VMEM((2, 2, H), jnp.float32)],
    )(ehat, e)[0]


def _edge_out_mm_body(ehat_ref, e_ref, cw, cb, out_ref, ce_ref, st_scr):
    i = pl.program_id(0)
    _edge_stats(ehat_ref, st_scr, i)

    @pl.when(i >= NB_E)
    def _():
        eo = _edge_norm(ehat_ref, e_ref, st_scr)
        out_ref[...] = eo
        ce = eo @ cw[...] + cb[...]
        ce_ref[0] = ce[:, :H]
        ce_ref[1] = ce[:, H:]


def _edge_out_mm(ehat, e, pnext):
    return pl.pallas_call(
        _edge_out_mm_body,
        grid=(2 * NB_E,),
        in_specs=[pl.BlockSpec((2, BE, H), lambda i: (0, i % NB_E, 0)),
                  pl.BlockSpec((BE, D), lambda i: (i % NB_E, 0)),
                  pl.BlockSpec((D, D), lambda i: (0, 0)),
                  pl.BlockSpec((1, D), lambda i: (0, 0))],
        out_specs=[pl.BlockSpec((BE, D), lambda i: (i % NB_E, 0)),
                   pl.BlockSpec((2, BE, H), lambda i: (0, i % NB_E, 0))],
        out_shape=[jax.ShapeDtypeStruct((E, D), jnp.float32),
                   jax.ShapeDtypeStruct((2, E, H), jnp.float32)],
        scratch_shapes=[pltpu.VMEM((2, 2, H), jnp.float32)],
    )(ehat, e, pnext["C_w"], pnext["C_b"].reshape(1, D))


def kernel(h, e, edge_index, params):
    src = edge_index[0]
    dst = edge_index[1]
    soff2 = jnp.concatenate([src, src + N])
    ce = _edge_mm(e, params[0])
    for li, p in enumerate(params):
        uh, tdst, tsrc = _node_mm(h, p)
        ehat2, numden = _sc_edge(
            tsrc.reshape(2 * N, D), tdst,
            ce.reshape(2 * E, H), soff2, dst)
        h = _node_out(uh, numden, h)
        if li + 1 < len(params):
            e, ce = _edge_out_mm(ehat2.reshape(2, E, H), e, params[li + 1])
        else:
            e = _edge_out(ehat2.reshape(2, E, H), e)
    return (h, e)
